# baseline (speedup 1.0000x reference)
import functools

import jax
import jax.numpy as jnp
from jax import lax
from jax.experimental import pallas as pl
from jax.experimental.pallas import tpu as pltpu
from jax.experimental.pallas import tpu_sc as plsc

N = 100000
D = 128
K = 50000
TILES = 16
NPAD = 100352
M = NPAD // TILES
VPT = M // 16
RBITS = 15
NBINS = 1 << RBITS
SHIFTS = (0, 15)
DOWN = NBINS // TILES
ROWS2D = NPAD // 128

NC = 2
NW = 32
CH = 128
FULL_CHUNKS = K // CH
REM = K - FULL_CHUNKS * CH


def _sort_body(keys_hbm, skeys_hbm, sidx_hbm,
               key_v, val_v, hist_v, rank_v, pos_v, part_v, pa_v, tot_v,
               kbuf0, vbuf0, kbuf1, vbuf1, hist_sh, part_sh, sem_s):
  tile = lax.axis_index("s")
  base = tile * M
  own_lo = tile * DOWN

  pltpu.sync_copy(keys_hbm.at[pl.ds(base, M)], key_v)

  def iloop(i, _):
    val_v[pl.ds(i * 16, 16)] = (
        lax.broadcasted_iota(jnp.int32, (16,), 0) + (base + i * 16))
    return 0
  lax.fori_loop(0, VPT, iloop, 0)

  def one_pass(shift, load_from, store_to):
    if load_from is not None:
      kb, vb = load_from
      pltpu.sync_copy(kb.at[pl.ds(base, M)], key_v)
      pltpu.sync_copy(vb.at[pl.ds(base, M)], val_v)

    with jax.named_scope("sort_hist"):
      def zloop(i, _):
        hist_v[pl.ds(i * 16, 16)] = jnp.zeros((16,), jnp.int32)
        return 0
      lax.fori_loop(0, NBINS // 16, zloop, 0)

      def hloop(i, _):
        sl = pl.ds(i * 16, 16)
        k = key_v[sl]
        d = (NBINS - 1) - (lax.shift_right_logical(k, shift) & (NBINS - 1))
        c, lm = plsc.scan_count(d)
        rank_v[sl] = plsc.load_gather(hist_v, [d]) + (c - 1)
        plsc.addupdate_scatter(hist_v, [d], c, mask=lm)
        return 0
      lax.fori_loop(0, VPT, hloop, 0, unroll=8)

    with jax.named_scope("sort_publish"):
      pltpu.sync_copy(hist_v, hist_sh.at[tile])
      plsc.subcore_barrier()

    with jax.named_scope("sort_offsets"):
      for t in range(TILES):
        pltpu.sync_copy(hist_sh.at[t, pl.ds(own_lo, DOWN)],
                        hist_v.at[pl.ds(t * DOWN, DOWN)])
      plsc.subcore_barrier()

      def caloop(j, acc):
        sl0 = j * 16
        run = jnp.zeros((16,), jnp.int32)
        for t in range(TILES):
          sl = pl.ds(t * DOWN + sl0, 16)
          h = hist_v[sl]
          hist_v[sl] = run
          run = run + h
        tot_v[pl.ds(sl0, 16)] = run
        return acc + run
      acc = lax.fori_loop(0, DOWN // 16, caloop, jnp.zeros((16,), jnp.int32))

      part_v[...] = lax.broadcast(jnp.sum(acc), (16,))
      pltpu.sync_copy(part_v, part_sh.at[pl.ds(tile * 16, 16)])
      plsc.subcore_barrier()
      pltpu.sync_copy(part_sh, pa_v)
      basev = jnp.zeros((16,), jnp.int32)
      for t in range(TILES):
        row = pa_v[pl.ds(t * 16, 16)]
        basev = basev + jnp.where(t < tile, row, jnp.zeros((16,), jnp.int32))

      def cbloop(j, carry):
        sl0 = j * 16
        v = tot_v[pl.ds(sl0, 16)]
        inc = plsc.cumsum(v)
        db = (inc - v) + carry
        for t in range(TILES):
          sl = pl.ds(t * DOWN + sl0, 16)
          hist_v[sl] = hist_v[sl] + db
        return carry + jnp.sum(v)
      lax.fori_loop(0, DOWN // 16, cbloop, basev)

      for r in range(TILES):
        pltpu.sync_copy(hist_v.at[pl.ds(r * DOWN, DOWN)],
                        hist_sh.at[r, pl.ds(own_lo, DOWN)])
      plsc.subcore_barrier()
      pltpu.sync_copy(hist_sh.at[tile], hist_v)

    dstk, dstv = store_to

    with jax.named_scope("sort_rank"):
      def ploop(j, _):
        rb = j * 128
        for u in range(8):
          sl = pl.ds(rb + u * 16, 16)
          k = key_v[sl]
          d = (NBINS - 1) - (lax.shift_right_logical(k, shift) & (NBINS - 1))
          pos_v[j, pl.ds(u * 16, 16)] = (
              plsc.load_gather(hist_v, [d]) + rank_v[sl])
        return 0
      lax.fori_loop(0, M // 128, ploop, 0, unroll=4)

    with jax.named_scope("sort_scatter"):
      descs = []
      for j in range(M // 128):
        rb = j * 128
        descs.append(pltpu.async_copy(
            key_v.at[pl.ds(rb, 128)], dstk.at[pos_v.at[j]], sem_s))
        descs.append(pltpu.async_copy(
            val_v.at[pl.ds(rb, 128)], dstv.at[pos_v.at[j]], sem_s))
      for dsc in descs:
        dsc.wait()
      plsc.subcore_barrier()
      plsc.subcore_barrier()

  one_pass(SHIFTS[0], None, (kbuf0, vbuf0))
  one_pass(SHIFTS[1], (kbuf0, vbuf0), (kbuf1, vbuf1))

  pltpu.sync_copy(kbuf1.at[pl.ds(base, M)], key_v)
  pltpu.sync_copy(key_v, skeys_hbm.at[pl.ds(base, M)])
  pltpu.sync_copy(vbuf1.at[pl.ds(base, M)], val_v)
  pltpu.sync_copy(val_v, sidx_hbm.at[pl.ds(base, M)])


_sort = functools.partial(
    pl.kernel,
    out_type=(
        jax.ShapeDtypeStruct((NPAD,), jnp.int32),
        jax.ShapeDtypeStruct((NPAD,), jnp.int32),
    ),
    mesh=plsc.VectorSubcoreMesh(
        core_axis_name="c", subcore_axis_name="s", num_cores=1),
    compiler_params=pltpu.CompilerParams(needs_layout_passes=False),
    scratch_types=[
        pltpu.VMEM((M,), jnp.int32),
        pltpu.VMEM((M,), jnp.int32),
        pltpu.VMEM((NBINS,), jnp.int32),
        pltpu.VMEM((M,), jnp.int32),
        pltpu.VMEM((M // 128, 128), jnp.int32),
        pltpu.VMEM((16,), jnp.int32),
        pltpu.VMEM((TILES * 16,), jnp.int32),
        pltpu.VMEM((DOWN,), jnp.int32),
        pltpu.VMEM_SHARED((NPAD,), jnp.int32),
        pltpu.VMEM_SHARED((NPAD,), jnp.int32),
        pltpu.VMEM_SHARED((NPAD,), jnp.int32),
        pltpu.VMEM_SHARED((NPAD,), jnp.int32),
        pltpu.VMEM_SHARED((TILES, NBINS), jnp.int32),
        pltpu.VMEM_SHARED((TILES * 16,), jnp.int32),
        pltpu.SemaphoreType.DMA,
    ],
)(_sort_body)


def _gather_body(x_hbm, sidx_hbm, skf_hbm, newx_hbm,
                 idx0_v, s0_v, rows0_v, idx1_v, s1_v, rows1_v,
                 idx80_v, s80_v, rows80_v, sem0, sem1, sem80):
  c = lax.axis_index("c")
  s = lax.axis_index("s")
  wid = s * NC + c
  bufs = ((idx0_v, s0_v, rows0_v, sem0), (idx1_v, s1_v, rows1_v, sem1))

  def scale_rows(rows_ref, scale_ref, nrows):
    def gloop(g, _):
      sv16 = scale_ref[pl.ds(g * 16, 16)]
      for r16 in range(16):
        r = g * 16 + r16
        bvec = lax.broadcast(sv16[r16], (16,))
        for u in range(8):
          sl = (r, pl.ds(u * 16, 16))
          rows_ref[sl] = rows_ref[sl] * bvec
      return 0
    lax.fori_loop(0, nrows // 16, gloop, 0)

  def start(j, t):
    idx_v, _, rows_v, sem = bufs[t]
    cid = wid + NW * j

    @pl.when(cid < FULL_CHUNKS)
    def _():
      b = cid * CH
      pltpu.sync_copy(sidx_hbm.at[pl.ds(b, CH)], idx_v)
      pltpu.async_copy(x_hbm.at[idx_v], rows_v, sem)

  def process(j, t):
    idx_v, s_v, rows_v, sem = bufs[t]
    cid = wid + NW * j

    @pl.when(cid < FULL_CHUNKS)
    def _():
      b = cid * CH
      pltpu.make_async_copy(x_hbm.at[idx_v], rows_v, sem).wait()
      pltpu.sync_copy(skf_hbm.at[pl.ds(b, CH)], s_v)
      scale_rows(rows_v, s_v, CH)
      pltpu.sync_copy(rows_v, newx_hbm.at[pl.ds(b, CH)])

    start(j + 2, t)

  start(0, 0)
  start(1, 1)

  def pair(jj, _):
    process(2 * jj, 0)
    process(2 * jj + 1, 1)
    return 0

  nslots = -(-FULL_CHUNKS // NW)
  npairs = -(-nslots // 2)
  lax.fori_loop(0, npairs, pair, 0)

  @pl.when(wid == FULL_CHUNKS % NW)
  def _():
    b = FULL_CHUNKS * CH
    pltpu.sync_copy(sidx_hbm.at[pl.ds(b, REM)], idx80_v)
    pltpu.sync_copy(skf_hbm.at[pl.ds(b, REM)], s80_v)
    pltpu.async_copy(x_hbm.at[idx80_v], rows80_v, sem80).wait()
    scale_rows(rows80_v, s80_v, REM)
    pltpu.sync_copy(rows80_v, newx_hbm.at[pl.ds(b, REM)])


_gather = functools.partial(
    pl.kernel,
    out_type=jax.ShapeDtypeStruct((K, D), jnp.float32),
    mesh=plsc.VectorSubcoreMesh(
        core_axis_name="c", subcore_axis_name="s", num_cores=NC),
    compiler_params=pltpu.CompilerParams(needs_layout_passes=False),
    scratch_types=[
        pltpu.VMEM((CH,), jnp.int32),
        pltpu.VMEM((CH,), jnp.float32),
        pltpu.VMEM((CH, D), jnp.float32),
        pltpu.VMEM((CH,), jnp.int32),
        pltpu.VMEM((CH,), jnp.float32),
        pltpu.VMEM((CH, D), jnp.float32),
        pltpu.VMEM((REM,), jnp.int32),
        pltpu.VMEM((REM,), jnp.float32),
        pltpu.VMEM((REM, D), jnp.float32),
        pltpu.SemaphoreType.DMA,
        pltpu.SemaphoreType.DMA,
        pltpu.SemaphoreType.DMA,
    ],
)(_gather_body)


def _loss_body(sk_ref, out_ref):
  sv = lax.bitcast_convert_type(sk_ref[...], jnp.float32)
  row = lax.broadcasted_iota(jnp.int32, (ROWS2D, 128), 0)
  col = lax.broadcasted_iota(jnp.int32, (ROWS2D, 128), 1)
  lin = row * 128 + col
  eps = 1e-08
  top = jnp.where(lin < K, jnp.log(sv + eps), 0.0)
  rest = jnp.where((lin >= K) & (lin < N), jnp.log((1.0 - sv) + eps), 0.0)
  out_ref[...] = jnp.reshape(-(jnp.sum(top) + jnp.sum(rest)) / N, (1, 1))


_loss = pl.pallas_call(
    _loss_body,
    out_shape=jax.ShapeDtypeStruct((1, 1), jnp.float32),
)


def kernel(lw_matrix_hidden_state_last, trainable_vector_pooling):
  x = lw_matrix_hidden_state_last
  v = trainable_vector_pooling
  norm2 = jnp.linalg.norm(v)
  scores = x @ (v / (norm2 + 1e-08))
  scores = (scores - scores.mean()) / (scores.std() + 1e-08)
  sig = jax.nn.sigmoid(scores)
  svec = sig.squeeze(-1)

  keys = lax.bitcast_convert_type(svec, jnp.int32)
  keys_pad = jnp.concatenate([keys, jnp.zeros((NPAD - N,), jnp.int32)])
  skeys, sidx = _sort(keys_pad)
  skf = lax.bitcast_convert_type(skeys, jnp.float32)
  new_x = _gather(x, sidx, skf)
  pool_loss = _loss(skeys.reshape(ROWS2D, 128))[0, 0]
  return (new_x, pool_loss)


# speedup vs baseline: 1.1506x; 1.1506x over previous
import functools

import jax
import jax.numpy as jnp
from jax import lax
from jax.experimental import pallas as pl
from jax.experimental.pallas import tpu as pltpu
from jax.experimental.pallas import tpu_sc as plsc

N = 100000
D = 128
K = 50000
TILES = 16
NPAD = 100352
M = NPAD // TILES
VPT = M // 16
RBITS = 10
NBINS = 1 << RBITS
SHIFTS = (0, 10, 20)
ROWS2D = NPAD // 128

NC = 2
NW = 32
CH = 128
FULL_CHUNKS = K // CH
REM = K - FULL_CHUNKS * CH


def _sort_body(keys_hbm, skeys_hbm, sidx_hbm,
               key_v, val_v, hist_v, histall_v, tot_v, below_v, off_v,
               rank_v, pos_v, kbuf0, vbuf0, kbuf1, vbuf1, hist_sh, sem_s):
  tile = lax.axis_index("s")
  base = tile * M

  pltpu.sync_copy(keys_hbm.at[pl.ds(base, M)], key_v)

  def iloop(i, _):
    val_v[pl.ds(i * 16, 16)] = (
        lax.broadcasted_iota(jnp.int32, (16,), 0) + (base + i * 16))
    return 0
  lax.fori_loop(0, VPT, iloop, 0)

  def one_pass(shift, load_from, store_to):
    if load_from is not None:
      kb, vb = load_from
      pltpu.sync_copy(kb.at[pl.ds(base, M)], key_v)
      pltpu.sync_copy(vb.at[pl.ds(base, M)], val_v)

    with jax.named_scope("sort_hist"):
      def zloop(i, _):
        hist_v[pl.ds(i * 16, 16)] = jnp.zeros((16,), jnp.int32)
        return 0
      lax.fori_loop(0, NBINS // 16, zloop, 0)

      def hloop(i, _):
        sl = pl.ds(i * 16, 16)
        k = key_v[sl]
        d = (NBINS - 1) - (lax.shift_right_logical(k, shift) & (NBINS - 1))
        c, lm = plsc.scan_count(d)
        rank_v[sl] = plsc.load_gather(hist_v, [d]) + (c - 1)
        plsc.addupdate_scatter(hist_v, [d], c, mask=lm)
        return 0
      lax.fori_loop(0, VPT, hloop, 0, unroll=8)

    with jax.named_scope("sort_publish"):
      pltpu.sync_copy(hist_v, hist_sh.at[tile])
      plsc.subcore_barrier()
      pltpu.sync_copy(hist_sh, histall_v)

    def cloop(j, _):
      sl = pl.ds(j * 16, 16)
      tot = jnp.zeros((16,), jnp.int32)
      below = jnp.zeros((16,), jnp.int32)
      for t in range(TILES):
        h = histall_v[t, sl]
        tot = tot + h
        below = below + jnp.where(t < tile, h, jnp.zeros((16,), jnp.int32))
      tot_v[sl] = tot
      below_v[sl] = below
      return 0

    with jax.named_scope("sort_offsets"):
      lax.fori_loop(0, NBINS // 16, cloop, 0)

      def sloop(j, carry):
        sl = pl.ds(j * 16, 16)
        v = tot_v[sl]
        inc = plsc.cumsum(v)
        off_v[sl] = (inc - v) + below_v[sl] + carry
        return carry + jnp.sum(v)
      lax.fori_loop(0, NBINS // 16, sloop, jnp.int32(0))

    dstk, dstv = store_to

    with jax.named_scope("sort_rank"):
      def ploop(j, _):
        rb = j * 128
        for u in range(8):
          sl = pl.ds(rb + u * 16, 16)
          k = key_v[sl]
          d = (NBINS - 1) - (lax.shift_right_logical(k, shift) & (NBINS - 1))
          pos_v[j, pl.ds(u * 16, 16)] = (
              plsc.load_gather(off_v, [d]) + rank_v[sl])
        return 0
      lax.fori_loop(0, M // 128, ploop, 0, unroll=4)

    with jax.named_scope("sort_scatter"):
      descs = []
      for j in range(M // 128):
        rb = j * 128
        descs.append(pltpu.async_copy(
            key_v.at[pl.ds(rb, 128)], dstk.at[pos_v.at[j]], sem_s))
        descs.append(pltpu.async_copy(
            val_v.at[pl.ds(rb, 128)], dstv.at[pos_v.at[j]], sem_s))
      for dsc in descs:
        dsc.wait()
      plsc.subcore_barrier()
      plsc.subcore_barrier()

  one_pass(SHIFTS[0], None, (kbuf0, vbuf0))
  one_pass(SHIFTS[1], (kbuf0, vbuf0), (kbuf1, vbuf1))
  one_pass(SHIFTS[2], (kbuf1, vbuf1), (kbuf0, vbuf0))

  pltpu.sync_copy(kbuf0.at[pl.ds(base, M)], key_v)
  pltpu.sync_copy(key_v, skeys_hbm.at[pl.ds(base, M)])
  pltpu.sync_copy(vbuf0.at[pl.ds(base, M)], val_v)
  pltpu.sync_copy(val_v, sidx_hbm.at[pl.ds(base, M)])


_sort = functools.partial(
    pl.kernel,
    out_type=(
        jax.ShapeDtypeStruct((NPAD,), jnp.int32),
        jax.ShapeDtypeStruct((NPAD,), jnp.int32),
    ),
    mesh=plsc.VectorSubcoreMesh(
        core_axis_name="c", subcore_axis_name="s", num_cores=1),
    compiler_params=pltpu.CompilerParams(needs_layout_passes=False),
    scratch_types=[
        pltpu.VMEM((M,), jnp.int32),
        pltpu.VMEM((M,), jnp.int32),
        pltpu.VMEM((NBINS,), jnp.int32),
        pltpu.VMEM((TILES, NBINS), jnp.int32),
        pltpu.VMEM((NBINS,), jnp.int32),
        pltpu.VMEM((NBINS,), jnp.int32),
        pltpu.VMEM((NBINS,), jnp.int32),
        pltpu.VMEM((M,), jnp.int32),
        pltpu.VMEM((M // 128, 128), jnp.int32),
        pltpu.VMEM_SHARED((NPAD,), jnp.int32),
        pltpu.VMEM_SHARED((NPAD,), jnp.int32),
        pltpu.VMEM_SHARED((NPAD,), jnp.int32),
        pltpu.VMEM_SHARED((NPAD,), jnp.int32),
        pltpu.VMEM_SHARED((TILES, NBINS), jnp.int32),
        pltpu.SemaphoreType.DMA,
    ],
)(_sort_body)


def _gather_body(x_hbm, sidx_hbm, skf_hbm, newx_hbm,
                 idx0_v, s0_v, rows0_v, idx1_v, s1_v, rows1_v,
                 idx80_v, s80_v, rows80_v, sem0, sem1, sem80):
  c = lax.axis_index("c")
  s = lax.axis_index("s")
  wid = s * NC + c
  bufs = ((idx0_v, s0_v, rows0_v, sem0), (idx1_v, s1_v, rows1_v, sem1))

  def scale_rows(rows_ref, scale_ref, nrows):
    def gloop(g, _):
      sv16 = scale_ref[pl.ds(g * 16, 16)]
      for r16 in range(16):
        r = g * 16 + r16
        bvec = lax.broadcast(sv16[r16], (16,))
        for u in range(8):
          sl = (r, pl.ds(u * 16, 16))
          rows_ref[sl] = rows_ref[sl] * bvec
      return 0
    lax.fori_loop(0, nrows // 16, gloop, 0)

  def start(j, t):
    idx_v, _, rows_v, sem = bufs[t]
    cid = wid + NW * j

    @pl.when(cid < FULL_CHUNKS)
    def _():
      b = cid * CH
      pltpu.sync_copy(sidx_hbm.at[pl.ds(b, CH)], idx_v)
      pltpu.async_copy(x_hbm.at[idx_v], rows_v, sem)

  def process(j, t):
    idx_v, s_v, rows_v, sem = bufs[t]
    cid = wid + NW * j

    @pl.when(cid < FULL_CHUNKS)
    def _():
      b = cid * CH
      pltpu.make_async_copy(x_hbm.at[idx_v], rows_v, sem).wait()
      pltpu.sync_copy(skf_hbm.at[pl.ds(b, CH)], s_v)
      scale_rows(rows_v, s_v, CH)
      pltpu.sync_copy(rows_v, newx_hbm.at[pl.ds(b, CH)])

    start(j + 2, t)

  start(0, 0)
  start(1, 1)

  def pair(jj, _):
    process(2 * jj, 0)
    process(2 * jj + 1, 1)
    return 0

  nslots = -(-FULL_CHUNKS // NW)
  npairs = -(-nslots // 2)
  lax.fori_loop(0, npairs, pair, 0)

  @pl.when(wid == FULL_CHUNKS % NW)
  def _():
    b = FULL_CHUNKS * CH
    pltpu.sync_copy(sidx_hbm.at[pl.ds(b, REM)], idx80_v)
    pltpu.sync_copy(skf_hbm.at[pl.ds(b, REM)], s80_v)
    pltpu.async_copy(x_hbm.at[idx80_v], rows80_v, sem80).wait()
    scale_rows(rows80_v, s80_v, REM)
    pltpu.sync_copy(rows80_v, newx_hbm.at[pl.ds(b, REM)])


_gather = functools.partial(
    pl.kernel,
    out_type=jax.ShapeDtypeStruct((K, D), jnp.float32),
    mesh=plsc.VectorSubcoreMesh(
        core_axis_name="c", subcore_axis_name="s", num_cores=NC),
    compiler_params=pltpu.CompilerParams(needs_layout_passes=False),
    scratch_types=[
        pltpu.VMEM((CH,), jnp.int32),
        pltpu.VMEM((CH,), jnp.float32),
        pltpu.VMEM((CH, D), jnp.float32),
        pltpu.VMEM((CH,), jnp.int32),
        pltpu.VMEM((CH,), jnp.float32),
        pltpu.VMEM((CH, D), jnp.float32),
        pltpu.VMEM((REM,), jnp.int32),
        pltpu.VMEM((REM,), jnp.float32),
        pltpu.VMEM((REM, D), jnp.float32),
        pltpu.SemaphoreType.DMA,
        pltpu.SemaphoreType.DMA,
        pltpu.SemaphoreType.DMA,
    ],
)(_gather_body)


def _loss_body(sk_ref, out_ref):
  sv = lax.bitcast_convert_type(sk_ref[...], jnp.float32)
  row = lax.broadcasted_iota(jnp.int32, (ROWS2D, 128), 0)
  col = lax.broadcasted_iota(jnp.int32, (ROWS2D, 128), 1)
  lin = row * 128 + col
  eps = 1e-08
  top = jnp.where(lin < K, jnp.log(sv + eps), 0.0)
  rest = jnp.where((lin >= K) & (lin < N), jnp.log((1.0 - sv) + eps), 0.0)
  out_ref[...] = jnp.reshape(-(jnp.sum(top) + jnp.sum(rest)) / N, (1, 1))


_loss = pl.pallas_call(
    _loss_body,
    out_shape=jax.ShapeDtypeStruct((1, 1), jnp.float32),
)


def kernel(lw_matrix_hidden_state_last, trainable_vector_pooling):
  x = lw_matrix_hidden_state_last
  v = trainable_vector_pooling
  norm2 = jnp.linalg.norm(v)
  scores = x @ (v / (norm2 + 1e-08))
  scores = (scores - scores.mean()) / (scores.std() + 1e-08)
  sig = jax.nn.sigmoid(scores)
  svec = sig.squeeze(-1)

  keys = lax.bitcast_convert_type(svec, jnp.int32)
  keys_pad = jnp.concatenate([keys, jnp.zeros((NPAD - N,), jnp.int32)])
  skeys, sidx = _sort(keys_pad)
  skf = lax.bitcast_convert_type(skeys, jnp.float32)
  new_x = _gather(x, sidx, skf)
  pool_loss = _loss(skeys.reshape(ROWS2D, 128))[0, 0]
  return (new_x, pool_loss)
